# baseline (speedup 1.0000x reference)
import functools
import jax
import jax.numpy as jnp
from jax import lax
from jax.experimental import pallas as pl
from jax.experimental.pallas import tpu as pltpu
from jax.experimental.pallas import tpu_sc as plsc

VOCAB = 1000
ROWS = 1024
SEQ = 50
PLANE = VOCAB * ROWS
TOTAL = SEQ * PLANE
ZCHUNK = 64000
NW = 32


def _sc_body(ids_hbm, mask_hbm, out_hbm, zbuf, ids_v, mask_v, addr_v,
             zsem, lsem, ssem):
    wid = lax.axis_index("s") * 2 + lax.axis_index("c")
    lo = wid * SEQ // NW
    hi = (wid + 1) * SEQ // NW

    def _zero_zbuf(k, _):
        zbuf[pl.ds(k * 16, 16)] = jnp.zeros((16,), jnp.float32)
        return _

    lax.fori_loop(0, ZCHUNK // 16, _zero_zbuf, 0)

    def _plane(p, _):
        off = p * PLANE
        zcopies = [
            pltpu.make_async_copy(
                zbuf, out_hbm.at[pl.ds(off + c * ZCHUNK, ZCHUNK)], zsem)
            for c in range(PLANE // ZCHUNK)
        ]
        for cp in zcopies:
            cp.start()

        pltpu.make_async_copy(ids_hbm.at[p], ids_v, lsem).start()
        pltpu.make_async_copy(mask_hbm.at[p], mask_v, lsem).start()
        pltpu.make_async_copy(ids_hbm.at[p], ids_v, lsem).wait()
        pltpu.make_async_copy(mask_hbm.at[p], mask_v, lsem).wait()

        def _addr(k, _):
            iv = ids_v[pl.ds(k * 16, 16)]
            lane = k * 16 + lax.iota(jnp.int32, 16)
            addr_v[pl.ds(k * 16, 16)] = (p * VOCAB + iv) * ROWS + lane
            return _

        lax.fori_loop(0, ROWS // 16, _addr, 0)

        for cp in zcopies:
            cp.wait()

        scopies = [
            pltpu.make_async_copy(
                mask_v.at[pl.ds(c * 128, 128)],
                out_hbm.at[addr_v.at[pl.ds(c * 128, 128)]],
                ssem)
            for c in range(ROWS // 128)
        ]
        for cp in scopies:
            cp.start()
        for cp in scopies:
            cp.wait()
        return _

    lax.fori_loop(lo, hi, _plane, 0)


def kernel(input_ids, attention_mask):
    ids_t = input_ids.astype(jnp.int32).T
    mask_t = attention_mask.astype(jnp.float32).T
    out_flat = pl.kernel(
        _sc_body,
        out_type=jax.ShapeDtypeStruct((TOTAL,), jnp.float32),
        mesh=plsc.VectorSubcoreMesh(core_axis_name="c", subcore_axis_name="s"),
        scratch_types=[
            pltpu.VMEM((ZCHUNK,), jnp.float32),
            pltpu.VMEM((ROWS,), jnp.int32),
            pltpu.VMEM((ROWS,), jnp.float32),
            pltpu.VMEM((ROWS,), jnp.int32),
            pltpu.SemaphoreType.DMA,
            pltpu.SemaphoreType.DMA,
            pltpu.SemaphoreType.DMA,
        ],
    )(ids_t, mask_t)
    return jnp.transpose(out_flat.reshape(SEQ, VOCAB, ROWS), (2, 0, 1))


# speedup vs baseline: 5.3282x; 5.3282x over previous
import jax
import jax.numpy as jnp
from jax.experimental import pallas as pl

VOCAB = 1000
ROWS = 1024
SEQ = 50


def _onehot_body(ids_ref, mask_ref, out_ref):
    ids = ids_ref[0]
    mask = mask_ref[0]
    iota_v = jax.lax.broadcasted_iota(jnp.int32, (VOCAB, ROWS), 0)
    out_ref[0] = jnp.where(iota_v == ids, mask, 0.0)


def kernel(input_ids, attention_mask):
    ids_t = input_ids.astype(jnp.int32).T.reshape(SEQ, 1, ROWS)
    mask_t = attention_mask.astype(jnp.float32).T.reshape(SEQ, 1, ROWS)
    out_t = pl.pallas_call(
        _onehot_body,
        grid=(SEQ,),
        in_specs=[
            pl.BlockSpec((1, 1, ROWS), lambda j: (j, 0, 0)),
            pl.BlockSpec((1, 1, ROWS), lambda j: (j, 0, 0)),
        ],
        out_specs=pl.BlockSpec((1, VOCAB, ROWS), lambda j: (j, 0, 0)),
        out_shape=jax.ShapeDtypeStruct((SEQ, VOCAB, ROWS), jnp.float32),
    )(ids_t, mask_t)
    return jnp.transpose(out_t, (2, 0, 1))
